# Initial kernel scaffold; baseline (speedup 1.0000x reference)
import jax
import jax.numpy as jnp
from jax.experimental import pallas as pl


def kernel(x, hyperedge_index, hyperedge_weight, hyperedge_attr, batch, W1, b1, W2, b2, W3, b3):
    raise NotImplementedError("write your pallas kernel here")


# speedup vs baseline: 3.8509x; 3.8509x over previous
import functools

import jax
import jax.numpy as jnp
from jax import lax
from jax.experimental import pallas as pl
from jax.experimental.pallas import tpu as pltpu
from jax.experimental.pallas import tpu_sc as plsc

N = 10000
E = 10000
NNZ = 320000
D = 128
NP = 10240
NC = 2
NS = 16
NW = NC * NS
PER_W = NNZ // NW
K = 64
CH = 160
PER_WP = CH * K
NNZP = NW * PER_WP
ROWS_PER_TILE = NP // NS

_f32 = jnp.float32
_i32 = jnp.int32

_MESH = plsc.VectorSubcoreMesh(core_axis_name="c", subcore_axis_name="s")


@functools.partial(
    pl.kernel,
    out_type=(
        jax.ShapeDtypeStruct((NW, NP // 16, 16), _f32),
        jax.ShapeDtypeStruct((NW, NP // 16, 16), _f32),
    ),
    mesh=_MESH,
    scratch_types=[
        pltpu.VMEM((PER_W,), _i32),
        pltpu.VMEM((PER_W,), _i32),
        pltpu.VMEM((E,), _f32),
        pltpu.VMEM((NP // 16, 16), _f32),
        pltpu.VMEM((NP // 16, 16), _f32),
    ],
    compiler_params=pltpu.CompilerParams(needs_layout_passes=False,
                                         use_tc_tiling_on_sc=False),
)
def _deg_kernel(nidx_hbm, eidx_hbm, hw_hbm, dg_out, bg_out,
                nidx_v, eidx_v, hw_v, dg_v, bg_v):
    cid = lax.axis_index("c")
    sid = lax.axis_index("s")
    wid = cid * NS + sid
    base = wid * PER_W
    pltpu.sync_copy(nidx_hbm.at[pl.ds(base, PER_W)], nidx_v)
    pltpu.sync_copy(eidx_hbm.at[pl.ds(base, PER_W)], eidx_v)
    pltpu.sync_copy(hw_hbm, hw_v)

    zeros16 = jnp.zeros((16,), _f32)

    def zero_body(i, carry):
        dg_v[i, :] = zeros16
        bg_v[i, :] = zeros16
        return carry

    lax.fori_loop(0, NP // 16, zero_body, 0)

    ones16 = jnp.ones((16,), _f32)

    def acc_body(i, carry):
        n16 = nidx_v[pl.ds(i * 16, 16)]
        e16 = eidx_v[pl.ds(i * 16, 16)]
        w16 = plsc.load_gather(hw_v, [e16])
        plsc.addupdate_scatter(dg_v, [n16 >> 4, n16 & 15], w16)
        plsc.addupdate_scatter(bg_v, [e16 >> 4, e16 & 15], ones16)
        return carry

    lax.fori_loop(0, PER_W // 16, acc_body, 0)

    pltpu.sync_copy(dg_v, dg_out.at[wid])
    pltpu.sync_copy(bg_v, bg_out.at[wid])


@functools.partial(
    pl.kernel,
    out_type=jax.ShapeDtypeStruct((NC, NP, D), _f32),
    mesh=_MESH,
    scratch_types=[
        pltpu.VMEM((CH, K), _i32),
        pltpu.VMEM((CH, K), _i32),
        pltpu.VMEM((K, D), _f32),
        pltpu.VMEM((K, D), _f32),
        pltpu.VMEM_SHARED((NP, D), _f32),
        pltpu.SemaphoreType.DMA,
        pltpu.SemaphoreType.DMA,
    ],
    compiler_params=pltpu.CompilerParams(needs_layout_passes=False,
                                         use_tc_tiling_on_sc=False),
)
def _seg_kernel(feat_hbm, gidx_hbm, sidx_hbm, zeros_hbm, out_hbm,
                gidx_v, sidx_v, rows_a, rows_b, acc_sh, sem_a, sem_b):
    cid = lax.axis_index("c")
    sid = lax.axis_index("s")
    wid = cid * NS + sid
    pltpu.sync_copy(gidx_hbm.at[wid], gidx_v)
    pltpu.sync_copy(sidx_hbm.at[wid], sidx_v)
    pltpu.sync_copy(zeros_hbm.at[pl.ds(sid * ROWS_PER_TILE, ROWS_PER_TILE)],
                    acc_sh.at[pl.ds(sid * ROWS_PER_TILE, ROWS_PER_TILE)])
    plsc.subcore_barrier()

    def pair_body(c2, carry):
        c = c2 * 2
        da = pltpu.async_copy(feat_hbm.at[gidx_v.at[c]], rows_a, sem_a)
        db = pltpu.async_copy(feat_hbm.at[gidx_v.at[c + 1]], rows_b, sem_b)
        da.wait()
        pltpu.sync_copy(rows_a, acc_sh.at[sidx_v.at[c]], add=True)
        db.wait()
        pltpu.sync_copy(rows_b, acc_sh.at[sidx_v.at[c + 1]], add=True)
        return carry

    lax.fori_loop(0, CH // 2, pair_body, 0)
    plsc.subcore_barrier()
    pltpu.sync_copy(acc_sh.at[pl.ds(sid * ROWS_PER_TILE, ROWS_PER_TILE)],
                    out_hbm.at[cid, pl.ds(sid * ROWS_PER_TILE, ROWS_PER_TILE)])


_BM = 2048


def _mm_body(x_ref, w_ref, o_ref):
    o_ref[...] = jnp.dot(x_ref[...], w_ref[...], preferred_element_type=_f32)


def _mm(xp, w):
    return pl.pallas_call(
        _mm_body,
        grid=(NP // _BM,),
        in_specs=[
            pl.BlockSpec((_BM, D), lambda i: (i, 0)),
            pl.BlockSpec((D, D), lambda i: (0, 0)),
        ],
        out_specs=pl.BlockSpec((_BM, D), lambda i: (i, 0)),
        out_shape=jax.ShapeDtypeStruct((NP, D), _f32),
    )(xp, w)


def _fuse_edge_body(p_ref, bgp_ref, o_ref):
    s = p_ref[0] + p_ref[1]
    bg = jnp.sum(bgp_ref[...], axis=0)
    binv = jnp.where(bg > 0, 1.0 / bg, 0.0)
    o_ref[...] = s * binv[:, None]


def _fuse_edge(p, bgp):
    return pl.pallas_call(
        _fuse_edge_body,
        grid=(NP // _BM,),
        in_specs=[
            pl.BlockSpec((NC, _BM, D), lambda i: (0, i, 0)),
            pl.BlockSpec((NW, _BM), lambda i: (0, i)),
        ],
        out_specs=pl.BlockSpec((_BM, D), lambda i: (i, 0)),
        out_shape=jax.ShapeDtypeStruct((NP, D), _f32),
    )(p, bgp)


def _fuse_node_mm_body(q_ref, dgp_ref, b_ref, w_ref, o_ref):
    s = q_ref[0] + q_ref[1]
    dg = jnp.sum(dgp_ref[...], axis=0)
    dinv = jnp.where(dg > 0, 1.0 / dg, 0.0)
    h = jnp.maximum(s * dinv[:, None] + b_ref[...], 0.0)
    o_ref[...] = jnp.dot(h, w_ref[...], preferred_element_type=_f32)


def _fuse_node_mm(q, dgp, b, w):
    return pl.pallas_call(
        _fuse_node_mm_body,
        grid=(NP // _BM,),
        in_specs=[
            pl.BlockSpec((NC, _BM, D), lambda i: (0, i, 0)),
            pl.BlockSpec((NW, _BM), lambda i: (0, i)),
            pl.BlockSpec((1, D), lambda i: (0, 0)),
            pl.BlockSpec((D, D), lambda i: (0, 0)),
        ],
        out_specs=pl.BlockSpec((_BM, D), lambda i: (i, 0)),
        out_shape=jax.ShapeDtypeStruct((NP, D), _f32),
    )(q, dgp, b.reshape(1, D), w)


def _fuse_node_body(q_ref, dgp_ref, b_ref, o_ref):
    s = q_ref[0] + q_ref[1]
    dg = jnp.sum(dgp_ref[...], axis=0)
    dinv = jnp.where(dg > 0, 1.0 / dg, 0.0)
    o_ref[...] = jnp.maximum(s * dinv[:, None] + b_ref[...], 0.0)


def _fuse_node(q, dgp, b):
    return pl.pallas_call(
        _fuse_node_body,
        grid=(NP // _BM,),
        in_specs=[
            pl.BlockSpec((NC, _BM, D), lambda i: (0, i, 0)),
            pl.BlockSpec((NW, _BM), lambda i: (0, i)),
            pl.BlockSpec((1, D), lambda i: (0, 0)),
        ],
        out_specs=pl.BlockSpec((_BM, D), lambda i: (i, 0)),
        out_shape=jax.ShapeDtypeStruct((NP, D), _f32),
    )(q, dgp, b.reshape(1, D))


def kernel(x, hyperedge_index, hyperedge_weight, hyperedge_attr, batch,
           W1, b1, W2, b2, W3, b3):
    n_i = hyperedge_index[0].astype(_i32)
    e_i = hyperedge_index[1].astype(_i32)

    pad_len = NNZP - NNZ
    g_pad = jnp.full((pad_len,), N, _i32)
    s_pad = N + (jnp.arange(pad_len, dtype=_i32) % (NP - N))
    gidx_n = jnp.concatenate([n_i, g_pad]).reshape(NW, CH, K)
    sidx_e = jnp.concatenate([e_i, s_pad]).reshape(NW, CH, K)
    gidx_e = jnp.concatenate([e_i, g_pad]).reshape(NW, CH, K)
    sidx_n = jnp.concatenate([n_i, s_pad]).reshape(NW, CH, K)

    dgp, bgp = _deg_kernel(n_i, e_i, hyperedge_weight)
    dgp = dgp.reshape(NW, NP)
    bgp = bgp.reshape(NW, NP)

    zeros = jnp.zeros((NP, D), _f32)
    xp = jnp.zeros((NP, D), _f32).at[:N].set(x)

    xw = _mm(xp, W1)
    for (bcur, wnext) in ((b1, W2), (b2, W3)):
        p = _seg_kernel(xw, gidx_n, sidx_e, zeros)
        ef = _fuse_edge(p, bgp)
        q = _seg_kernel(ef, gidx_e, sidx_n, zeros)
        xw = _fuse_node_mm(q, dgp, bcur, wnext)
    p = _seg_kernel(xw, gidx_n, sidx_e, zeros)
    ef = _fuse_edge(p, bgp)
    q = _seg_kernel(ef, gidx_e, sidx_n, zeros)
    h = _fuse_node(q, dgp, b3)
    return h[:N]
